# Initial kernel scaffold; baseline (speedup 1.0000x reference)
import jax
import jax.numpy as jnp
from jax.experimental import pallas as pl


def kernel(h, edge_index, W, b):
    raise NotImplementedError("write your pallas kernel here")


# speedup vs baseline: 4.4230x; 4.4230x over previous
import functools

import jax
import jax.numpy as jnp
from jax import lax
from jax.experimental import pallas as pl
from jax.experimental.pallas import tpu as pltpu
from jax.experimental.pallas import tpu_sc as plsc

N_NODES = 10000
N_EDGES = 160000
D = 256
DH = 128

NC = 2
NS = 16
L = 16
NW = NC * NS

NPAD = 10240
SEG = NPAD // NS
HIST = NPAD + L

CHUNK = 128
EPAD = NS * 80 * CHUNK
EDGES_PER_TILE = EPAD // NS
N_CHUNKS = EDGES_PER_TILE // CHUNK
DUMMY = NPAD
ACC_ROWS = NPAD + CHUNK
ZCH = ACC_ROWS // CHUNK
OUT_CH = (NPAD // NS) // CHUNK

_MESH = plsc.VectorSubcoreMesh(core_axis_name="c", subcore_axis_name="s")


@functools.partial(
    pl.kernel,
    out_type=jax.ShapeDtypeStruct((NPAD,), jnp.float32),
    mesh=_MESH,
    compiler_params=pltpu.CompilerParams(needs_layout_passes=False),
    scratch_types=[
        pltpu.VMEM((HIST,), jnp.float32),
        pltpu.VMEM((EDGES_PER_TILE,), jnp.int32),
        pltpu.VMEM((SEG,), jnp.float32),
        pltpu.VMEM((SEG,), jnp.float32),
        pltpu.VMEM_SHARED((NS, HIST), jnp.float32),
    ],
)
def _norm_kernel(dst_hbm, norm_hbm, hist_v, dst_v, seg_acc, seg_tmp, stage_sh):
    sid = lax.axis_index("s")
    cid = lax.axis_index("c")
    zeros = jnp.zeros((L,), jnp.float32)
    ones = jnp.ones((L,), jnp.float32)

    def zbody(i, c):
        hist_v[pl.ds(i * L, L)] = zeros
        return c

    lax.fori_loop(0, HIST // L, zbody, 0)

    pltpu.sync_copy(dst_hbm.at[pl.ds(sid * EDGES_PER_TILE, EDGES_PER_TILE)], dst_v)

    def hbody(i, c):
        idx = dst_v[pl.ds(i * L, L)]
        plsc.addupdate_scatter(hist_v, [idx], ones)
        return c

    lax.fori_loop(0, EDGES_PER_TILE // L, hbody, 0)

    pltpu.sync_copy(hist_v, stage_sh.at[sid])
    plsc.subcore_barrier()

    pltpu.sync_copy(stage_sh.at[0, pl.ds(sid * SEG, SEG)], seg_acc)
    for k in range(1, NS):
        pltpu.sync_copy(stage_sh.at[k, pl.ds(sid * SEG, SEG)], seg_tmp)

        def abody(i, c, _k=k):
            s = pl.ds(i * L, L)
            seg_acc[s] = seg_acc[s] + seg_tmp[s]
            return c

        lax.fori_loop(0, SEG // L, abody, 0)

    def nbody(i, c):
        s = pl.ds(i * L, L)
        x = seg_acc[s]
        xi = lax.bitcast_convert_type(x, jnp.int32)
        y = lax.bitcast_convert_type(jnp.int32(0x5F3759DF) - (xi >> 1), jnp.float32)
        y = y * (1.5 - 0.5 * x * y * y)
        y = y * (1.5 - 0.5 * x * y * y)
        y = y * (1.5 - 0.5 * x * y * y)
        seg_acc[s] = jnp.where(x > 0.5, y, 0.0)
        return c

    lax.fori_loop(0, SEG // L, nbody, 0)

    @pl.when(sid // (NS // NC) == cid)
    def _():
        pltpu.sync_copy(seg_acc, norm_hbm.at[pl.ds(sid * SEG, SEG)])


@functools.partial(
    pl.kernel,
    out_type=[
        jax.ShapeDtypeStruct((NPAD, DH), jnp.float32),
        jax.ShapeDtypeStruct((NPAD, DH), jnp.float32),
    ],
    mesh=_MESH,
    compiler_params=pltpu.CompilerParams(needs_layout_passes=False),
    scratch_types=[
        pltpu.VMEM((CHUNK,), jnp.int32),
        pltpu.VMEM((CHUNK,), jnp.int32),
        pltpu.VMEM((CHUNK, DH), jnp.float32),
        pltpu.VMEM_SHARED((ACC_ROWS, DH), jnp.float32),
        pltpu.SemaphoreType.DMA,
    ],
)
def _agg_kernel(hwn0_hbm, hwn1_hbm, src_hbm, dst_hbm, out0_hbm, out1_hbm,
                src_v, dst_v, rows_v, acc_sh, sem):
    sid = lax.axis_index("s")
    cid = lax.axis_index("c")
    zeros = jnp.zeros((L,), jnp.float32)

    def zrow(i, c):
        for j in range(DH // L):
            rows_v[i, pl.ds(j * L, L)] = zeros
        return c

    lax.fori_loop(0, CHUNK, zrow, 0)

    def zacc(k, c):
        ch = sid + k * NS

        @pl.when(ch < ZCH)
        def _():
            pltpu.sync_copy(rows_v, acc_sh.at[pl.ds(ch * CHUNK, CHUNK)])

        return c

    lax.fori_loop(0, (ZCH + NS - 1) // NS, zacc, 0)
    plsc.subcore_barrier()

    ebase = sid * EDGES_PER_TILE

    def edge_loop(hwn_ref):
        def ebody(ch, c):
            off = ebase + ch * CHUNK
            pltpu.sync_copy(src_hbm.at[pl.ds(off, CHUNK)], src_v)
            pltpu.sync_copy(dst_hbm.at[pl.ds(off, CHUNK)], dst_v)
            pltpu.async_copy(hwn_ref.at[src_v], rows_v, sem).wait()
            pltpu.sync_copy(rows_v, acc_sh.at[dst_v], add=True)
            return c

        lax.fori_loop(0, N_CHUNKS, ebody, 0)

    @pl.when(cid == 0)
    def _():
        edge_loop(hwn0_hbm)

    @pl.when(cid == 1)
    def _():
        edge_loop(hwn1_hbm)

    plsc.subcore_barrier()

    row0 = sid * (NPAD // NS)

    def cbody(k, c):
        r = row0 + k * CHUNK
        pltpu.sync_copy(acc_sh.at[pl.ds(r, CHUNK)], rows_v)

        @pl.when(cid == 0)
        def _():
            pltpu.sync_copy(rows_v, out0_hbm.at[pl.ds(r, CHUNK)])

        @pl.when(cid == 1)
        def _():
            pltpu.sync_copy(rows_v, out1_hbm.at[pl.ds(r, CHUNK)])

        return c

    lax.fori_loop(0, OUT_CH, cbody, 0)


ROWB = 1000


def _mm_body(h_ref, w_ref, n_ref, o0_ref, o1_ref):
    hw = (
        jnp.dot(h_ref[...], w_ref[...], preferred_element_type=jnp.float32)
        * n_ref[...]
    )
    o0_ref[...] = hw[:, :DH]
    o1_ref[...] = hw[:, DH:]


def _matmul_norm(h, W, norm2d):
    return pl.pallas_call(
        _mm_body,
        grid=(N_NODES // ROWB,),
        in_specs=[
            pl.BlockSpec((ROWB, D), lambda i: (i, 0)),
            pl.BlockSpec((D, D), lambda i: (0, 0)),
            pl.BlockSpec((ROWB, 1), lambda i: (i, 0)),
        ],
        out_specs=[
            pl.BlockSpec((ROWB, DH), lambda i: (i, 0)),
            pl.BlockSpec((ROWB, DH), lambda i: (i, 0)),
        ],
        out_shape=[
            jax.ShapeDtypeStruct((N_NODES, DH), jnp.float32),
            jax.ShapeDtypeStruct((N_NODES, DH), jnp.float32),
        ],
    )(h, W, norm2d)


def _fin_body(a0_ref, a1_ref, n_ref, b_ref, o_ref):
    a = jnp.concatenate([a0_ref[...], a1_ref[...]], axis=1)
    o_ref[...] = jnp.maximum(a * n_ref[...] + b_ref[...], 0.0)


def _finalize(agg0, agg1, norm2d, b2d):
    return pl.pallas_call(
        _fin_body,
        grid=(N_NODES // ROWB,),
        in_specs=[
            pl.BlockSpec((ROWB, DH), lambda i: (i, 0)),
            pl.BlockSpec((ROWB, DH), lambda i: (i, 0)),
            pl.BlockSpec((ROWB, 1), lambda i: (i, 0)),
            pl.BlockSpec((1, D), lambda i: (0, 0)),
        ],
        out_specs=pl.BlockSpec((ROWB, D), lambda i: (i, 0)),
        out_shape=jax.ShapeDtypeStruct((N_NODES, D), jnp.float32),
    )(agg0, agg1, norm2d, b2d)


def kernel(h, edge_index, W, b):
    src = edge_index[0].astype(jnp.int32)
    dst = edge_index[1].astype(jnp.int32)
    src_p = jnp.pad(src, (0, EPAD - N_EDGES), constant_values=0)
    dst_p = jnp.pad(dst, (0, EPAD - N_EDGES), constant_values=DUMMY)

    norm_pad = _norm_kernel(dst_p)
    norm2d = norm_pad[:N_NODES, None]
    hwn0, hwn1 = _matmul_norm(h, W, norm2d)
    agg0, agg1 = _agg_kernel(hwn0, hwn1, src_p, dst_p)
    return _finalize(agg0[:N_NODES], agg1[:N_NODES], norm2d, b[None, :])


# speedup vs baseline: 5.9245x; 1.3395x over previous
import functools

import jax
import jax.numpy as jnp
from jax import lax
from jax.experimental import pallas as pl
from jax.experimental.pallas import tpu as pltpu
from jax.experimental.pallas import tpu_sc as plsc

N_NODES = 10000
N_EDGES = 160000
D = 256
DH = 128

NC = 2
NS = 16
L = 16
NW = NC * NS

NPAD = 10240
SEG = NPAD // NS
HIST = NPAD + L

CHUNK = 128
EPAD = NS * 80 * CHUNK
EDGES_PER_TILE = EPAD // NS
N_CHUNKS = EDGES_PER_TILE // CHUNK
DUMMY = NPAD
ACC_ROWS = NPAD + CHUNK
ZCH = ACC_ROWS // CHUNK
OUT_CH = (NPAD // NS) // CHUNK

_MESH = plsc.VectorSubcoreMesh(core_axis_name="c", subcore_axis_name="s")


@functools.partial(
    pl.kernel,
    out_type=jax.ShapeDtypeStruct((NPAD,), jnp.float32),
    mesh=_MESH,
    compiler_params=pltpu.CompilerParams(needs_layout_passes=False),
    scratch_types=[
        pltpu.VMEM((HIST,), jnp.float32),
        pltpu.VMEM((EDGES_PER_TILE,), jnp.int32),
        pltpu.VMEM((SEG,), jnp.float32),
        pltpu.VMEM((SEG,), jnp.float32),
        pltpu.VMEM_SHARED((NS, HIST), jnp.float32),
    ],
)
def _norm_kernel(dst_hbm, norm_hbm, hist_v, dst_v, seg_acc, seg_tmp, stage_sh):
    sid = lax.axis_index("s")
    cid = lax.axis_index("c")
    zeros = jnp.zeros((L,), jnp.float32)
    ones = jnp.ones((L,), jnp.float32)

    def zbody(i, c):
        hist_v[pl.ds(i * L, L)] = zeros
        return c

    lax.fori_loop(0, HIST // L, zbody, 0)

    pltpu.sync_copy(dst_hbm.at[pl.ds(sid * EDGES_PER_TILE, EDGES_PER_TILE)], dst_v)

    def hbody(i, c):
        idx = dst_v[pl.ds(i * L, L)]
        plsc.addupdate_scatter(hist_v, [idx], ones)
        return c

    lax.fori_loop(0, EDGES_PER_TILE // L, hbody, 0)

    pltpu.sync_copy(hist_v, stage_sh.at[sid])
    plsc.subcore_barrier()

    pltpu.sync_copy(stage_sh.at[0, pl.ds(sid * SEG, SEG)], seg_acc)
    for k in range(1, NS):
        pltpu.sync_copy(stage_sh.at[k, pl.ds(sid * SEG, SEG)], seg_tmp)

        def abody(i, c, _k=k):
            s = pl.ds(i * L, L)
            seg_acc[s] = seg_acc[s] + seg_tmp[s]
            return c

        lax.fori_loop(0, SEG // L, abody, 0)

    def nbody(i, c):
        s = pl.ds(i * L, L)
        x = seg_acc[s]
        xi = lax.bitcast_convert_type(x, jnp.int32)
        y = lax.bitcast_convert_type(jnp.int32(0x5F3759DF) - (xi >> 1), jnp.float32)
        y = y * (1.5 - 0.5 * x * y * y)
        y = y * (1.5 - 0.5 * x * y * y)
        y = y * (1.5 - 0.5 * x * y * y)
        seg_acc[s] = jnp.where(x > 0.5, y, 0.0)
        return c

    lax.fori_loop(0, SEG // L, nbody, 0)

    @pl.when(sid // (NS // NC) == cid)
    def _():
        pltpu.sync_copy(seg_acc, norm_hbm.at[pl.ds(sid * SEG, SEG)])


@functools.partial(
    pl.kernel,
    out_type=[
        jax.ShapeDtypeStruct((NPAD, DH), jnp.float32),
        jax.ShapeDtypeStruct((NPAD, DH), jnp.float32),
    ],
    mesh=_MESH,
    compiler_params=pltpu.CompilerParams(needs_layout_passes=False),
    scratch_types=[
        pltpu.VMEM((N_CHUNKS // 2, CHUNK), jnp.int32),
        pltpu.VMEM((N_CHUNKS // 2, CHUNK), jnp.int32),
        pltpu.VMEM((CHUNK, DH), jnp.float32),
        pltpu.VMEM((CHUNK, DH), jnp.float32),
        pltpu.VMEM_SHARED((ACC_ROWS, DH), jnp.float32),
        pltpu.SemaphoreType.DMA,
        pltpu.SemaphoreType.DMA,
    ],
)
def _agg_kernel(hwn0_hbm, hwn1_hbm, src_hbm, dst_hbm, out0_hbm, out1_hbm,
                src_v, dst_v, rows0_v, rows1_v, acc_sh, sem0, sem1):
    sid = lax.axis_index("s")
    cid = lax.axis_index("c")
    zeros = jnp.zeros((L,), jnp.float32)

    def zrow(i, c):
        for j in range(DH // L):
            rows0_v[i, pl.ds(j * L, L)] = zeros
        return c

    lax.fori_loop(0, CHUNK, zrow, 0)

    def zacc(k, c):
        ch = sid + k * NS

        @pl.when(ch < ZCH)
        def _():
            pltpu.sync_copy(rows0_v, acc_sh.at[pl.ds(ch * CHUNK, CHUNK)])

        return c

    lax.fori_loop(0, (ZCH + NS - 1) // NS, zacc, 0)
    plsc.subcore_barrier()

    def edge_loop(hwn_ref):
        def wait0():
            pltpu.make_async_copy(hwn_ref.at[pl.ds(0, CHUNK)], rows0_v, sem0).wait()

        def wait1():
            pltpu.make_async_copy(hwn_ref.at[pl.ds(0, CHUNK)], rows1_v, sem1).wait()

        half_chunks = N_CHUNKS // 2
        for half in range(2):
            base = sid * N_CHUNKS + half * half_chunks
            pltpu.sync_copy(src_hbm.at[pl.ds(base, half_chunks)], src_v)
            pltpu.sync_copy(dst_hbm.at[pl.ds(base, half_chunks)], dst_v)

            pltpu.async_copy(hwn_ref.at[src_v.at[0]], rows0_v, sem0)

            def ebody(g, c):
                c0 = g * 2
                pltpu.async_copy(hwn_ref.at[src_v.at[c0 + 1]], rows1_v, sem1)
                wait0()
                pltpu.sync_copy(rows0_v, acc_sh.at[dst_v.at[c0]], add=True)

                @pl.when(c0 + 2 < half_chunks)
                def _():
                    pltpu.async_copy(hwn_ref.at[src_v.at[c0 + 2]], rows0_v, sem0)

                wait1()
                pltpu.sync_copy(rows1_v, acc_sh.at[dst_v.at[c0 + 1]], add=True)
                return c

            lax.fori_loop(0, half_chunks // 2, ebody, 0)

    @pl.when(cid == 0)
    def _():
        edge_loop(hwn0_hbm)

    @pl.when(cid == 1)
    def _():
        edge_loop(hwn1_hbm)

    plsc.subcore_barrier()

    row0 = sid * (NPAD // NS)

    def cbody(k, c):
        r = row0 + k * CHUNK
        pltpu.sync_copy(acc_sh.at[pl.ds(r, CHUNK)], rows0_v)

        @pl.when(cid == 0)
        def _():
            pltpu.sync_copy(rows0_v, out0_hbm.at[pl.ds(r, CHUNK)])

        @pl.when(cid == 1)
        def _():
            pltpu.sync_copy(rows0_v, out1_hbm.at[pl.ds(r, CHUNK)])

        return c

    lax.fori_loop(0, OUT_CH, cbody, 0)


ROWB = 1000


def _mm_body(h_ref, w_ref, n_ref, o0_ref, o1_ref):
    hw = (
        jnp.dot(h_ref[...], w_ref[...], preferred_element_type=jnp.float32)
        * n_ref[...]
    )
    o0_ref[...] = hw[:, :DH]
    o1_ref[...] = hw[:, DH:]


def _matmul_norm(h, W, norm2d):
    return pl.pallas_call(
        _mm_body,
        grid=(N_NODES // ROWB,),
        in_specs=[
            pl.BlockSpec((ROWB, D), lambda i: (i, 0)),
            pl.BlockSpec((D, D), lambda i: (0, 0)),
            pl.BlockSpec((ROWB, 1), lambda i: (i, 0)),
        ],
        out_specs=[
            pl.BlockSpec((ROWB, DH), lambda i: (i, 0)),
            pl.BlockSpec((ROWB, DH), lambda i: (i, 0)),
        ],
        out_shape=[
            jax.ShapeDtypeStruct((N_NODES, DH), jnp.float32),
            jax.ShapeDtypeStruct((N_NODES, DH), jnp.float32),
        ],
    )(h, W, norm2d)


def _fin_body(a0_ref, a1_ref, n_ref, b_ref, o_ref):
    a = jnp.concatenate([a0_ref[...], a1_ref[...]], axis=1)
    o_ref[...] = jnp.maximum(a * n_ref[...] + b_ref[...], 0.0)


def _finalize(agg0, agg1, norm2d, b2d):
    return pl.pallas_call(
        _fin_body,
        grid=(N_NODES // ROWB,),
        in_specs=[
            pl.BlockSpec((ROWB, DH), lambda i: (i, 0)),
            pl.BlockSpec((ROWB, DH), lambda i: (i, 0)),
            pl.BlockSpec((ROWB, 1), lambda i: (i, 0)),
            pl.BlockSpec((1, D), lambda i: (0, 0)),
        ],
        out_specs=pl.BlockSpec((ROWB, D), lambda i: (i, 0)),
        out_shape=jax.ShapeDtypeStruct((N_NODES, D), jnp.float32),
    )(agg0, agg1, norm2d, b2d)


def kernel(h, edge_index, W, b):
    src = edge_index[0].astype(jnp.int32)
    dst = edge_index[1].astype(jnp.int32)
    src_p = jnp.pad(src, (0, EPAD - N_EDGES), constant_values=0)
    dst_p = jnp.pad(dst, (0, EPAD - N_EDGES), constant_values=DUMMY)

    norm_pad = _norm_kernel(dst_p)
    norm2d = norm_pad[:N_NODES, None]
    hwn0, hwn1 = _matmul_norm(h, W, norm2d)
    src2 = src_p.reshape(NS * N_CHUNKS, CHUNK)
    dst2 = dst_p.reshape(NS * N_CHUNKS, CHUNK)
    agg0, agg1 = _agg_kernel(hwn0, hwn1, src2, dst2)
    return _finalize(agg0[:N_NODES], agg1[:N_NODES], norm2d, b[None, :])
